# baseline (speedup 1.0000x reference)
import functools

import jax
import jax.numpy as jnp
from jax.experimental import pallas as pl
from jax.experimental.pallas import tpu as pltpu

B = 16384
STATE_SIZE = 64
REWARD_SIZE = 2
IN_DIM = STATE_SIZE + REWARD_SIZE
HIDDEN = IN_DIM * 40
ACTION_SIZE = 1024
QCOLS = ACTION_SIZE * REWARD_SIZE

BLK = 512
NB = B // BLK


def _select(q, p0, p1):
    lane = jax.lax.broadcasted_iota(jnp.int32, (1, QCOLS), 1)
    even = (lane & 1) == 0
    evenlane = lane & -2
    par_f = (lane & 1).astype(jnp.float32)
    w_il = jnp.where(even, p0, p1)
    pp = q * w_il
    pairsum = pp + pltpu.roll(pp, shift=QCOLS - 1, axis=1)
    prodm = jnp.where(even, pairsum, -jnp.inf)
    j = jnp.argmax(prodm, axis=1).astype(jnp.int32)[:, None]
    s = jnp.where(evenlane == j, q, 0.0)
    hq1 = jnp.sum(s * par_f, axis=1, keepdims=True)
    hq0 = jnp.sum(s, axis=1, keepdims=True) - hq1
    return jnp.concatenate([hq0, hq1], axis=1)


def _fused_kernel(x_ref, w1_ref, b1_ref, w2_ref, b2_ref, q_ref, hq_ref,
                  hql_ref, q_scr, p_scr):
    i = pl.program_id(0)

    hq_ref[...] = _select(q_scr[...], p_scr[:, 0:1], p_scr[:, 1:2])

    x = x_ref[...]
    h = jnp.dot(x, w1_ref[...], preferred_element_type=jnp.float32)
    h = jnp.maximum(h + b1_ref[...], 0.0)
    qm = jnp.dot(h, w2_ref[...], preferred_element_type=jnp.float32)
    qm = qm + b2_ref[...]
    q_ref[...] = qm
    q_scr[...] = qm
    p_scr[...] = x[:, STATE_SIZE:]

    @pl.when(i == NB - 1)
    def _last():
        hql_ref[...] = _select(qm, x[:, STATE_SIZE:STATE_SIZE + 1],
                               x[:, STATE_SIZE + 1:STATE_SIZE + 2])


@functools.partial(jax.jit, static_argnames=())
def kernel(state, preference, W1, b1, W2, b2):
    x = jnp.concatenate([state, preference], axis=1)
    w1t = W1.T
    w2t = W2.T
    b1r = b1.reshape(1, HIDDEN)
    b2r = b2.reshape(1, QCOLS)
    grid = (NB,)
    q, hq, hql = pl.pallas_call(
        _fused_kernel,
        grid=grid,
        in_specs=[
            pl.BlockSpec((BLK, IN_DIM), lambda i: (i, 0)),
            pl.BlockSpec((IN_DIM, HIDDEN), lambda i: (0, 0)),
            pl.BlockSpec((1, HIDDEN), lambda i: (0, 0)),
            pl.BlockSpec((HIDDEN, QCOLS), lambda i: (0, 0)),
            pl.BlockSpec((1, QCOLS), lambda i: (0, 0)),
        ],
        out_specs=[
            pl.BlockSpec((BLK, QCOLS), lambda i: (i, 0)),
            pl.BlockSpec((BLK, REWARD_SIZE), lambda i: (jnp.maximum(i - 1, 0), 0)),
            pl.BlockSpec((BLK, REWARD_SIZE), lambda i: (0, 0)),
        ],
        out_shape=[
            jax.ShapeDtypeStruct((B, QCOLS), jnp.float32),
            jax.ShapeDtypeStruct((B, REWARD_SIZE), jnp.float32),
            jax.ShapeDtypeStruct((BLK, REWARD_SIZE), jnp.float32),
        ],
        scratch_shapes=[
            pltpu.VMEM((BLK, QCOLS), jnp.float32),
            pltpu.VMEM((BLK, REWARD_SIZE), jnp.float32),
        ],
        compiler_params=pltpu.CompilerParams(
            dimension_semantics=("arbitrary",),
        ),
    )(x, w1t, b1r, w2t, b2r)
    hq_full = jnp.concatenate([hq[:B - BLK], hql], axis=0)
    return hq_full, q.reshape(B, ACTION_SIZE, REWARD_SIZE)


# speedup vs baseline: 1.0308x; 1.0308x over previous
import functools

import jax
import jax.numpy as jnp
from jax.experimental import pallas as pl
from jax.experimental.pallas import tpu as pltpu

B = 16384
STATE_SIZE = 64
REWARD_SIZE = 2
IN_DIM = STATE_SIZE + REWARD_SIZE
HIDDEN = IN_DIM * 40
ACTION_SIZE = 1024
QCOLS = ACTION_SIZE * REWARD_SIZE

BLK = 512


def _fused_kernel(x_ref, w1_ref, b1_ref, w2_ref, b2_ref, q_ref, hq_ref):
    x = x_ref[...]
    h = jnp.dot(x, w1_ref[...], preferred_element_type=jnp.float32)
    h = jnp.maximum(h + b1_ref[...], 0.0)
    q = jnp.dot(h, w2_ref[...], preferred_element_type=jnp.float32)
    q = q + b2_ref[...]
    q_ref[...] = q

    p0 = x[:, STATE_SIZE:STATE_SIZE + 1]
    p1 = x[:, STATE_SIZE + 1:STATE_SIZE + 2]
    lane = jax.lax.broadcasted_iota(jnp.int32, (1, QCOLS), 1)
    even = (lane & 1) == 0
    evenlane = lane & -2
    par_f = (lane & 1).astype(jnp.float32)
    w_il = jnp.where(even, p0, p1)
    pp = q * w_il
    pairsum = pp + pltpu.roll(pp, shift=QCOLS - 1, axis=1)
    prodm = jnp.where(even, pairsum, -jnp.inf)
    j = jnp.argmax(prodm, axis=1).astype(jnp.int32)[:, None]
    s = jnp.where(evenlane == j, q, 0.0)
    hq1 = jnp.sum(s * par_f, axis=1, keepdims=True)
    hq0 = jnp.sum(s, axis=1, keepdims=True) - hq1
    hq_ref[...] = jnp.concatenate([hq0, hq1], axis=1)


@functools.partial(jax.jit, static_argnames=())
def kernel(state, preference, W1, b1, W2, b2):
    x = jnp.concatenate([state, preference], axis=1)
    w1t = W1.T
    w2t = W2.T
    b1r = b1.reshape(1, HIDDEN)
    b2r = b2.reshape(1, QCOLS)
    grid = (B // BLK,)
    q, hq = pl.pallas_call(
        _fused_kernel,
        grid=grid,
        in_specs=[
            pl.BlockSpec((BLK, IN_DIM), lambda i: (i, 0)),
            pl.BlockSpec((IN_DIM, HIDDEN), lambda i: (0, 0)),
            pl.BlockSpec((1, HIDDEN), lambda i: (0, 0)),
            pl.BlockSpec((HIDDEN, QCOLS), lambda i: (0, 0)),
            pl.BlockSpec((1, QCOLS), lambda i: (0, 0)),
        ],
        out_specs=[
            pl.BlockSpec((BLK, QCOLS), lambda i: (i, 0)),
            pl.BlockSpec((BLK, REWARD_SIZE), lambda i: (i, 0)),
        ],
        out_shape=[
            jax.ShapeDtypeStruct((B, QCOLS), jnp.float32),
            jax.ShapeDtypeStruct((B, REWARD_SIZE), jnp.float32),
        ],
        compiler_params=pltpu.CompilerParams(
            dimension_semantics=("arbitrary",),
        ),
    )(x, w1t, b1r, w2t, b2r)
    return hq, q.reshape(B, ACTION_SIZE, REWARD_SIZE)
